# baseline (speedup 1.0000x reference)
import functools

import jax
import jax.numpy as jnp
from jax import lax
from jax.experimental import pallas as pl
from jax.experimental.pallas import tpu as pltpu
from jax.experimental.pallas import tpu_sc as plsc

_NC = 2
_NS = 16
_K = 128
_G = 128
_NB = 2
_NI = 4


def _make_agg(NP, D, CH):
    DH = D // _NC
    rows_per_sub = NP // _NS
    n_out_chunks = rows_per_sub // _K
    mesh = plsc.VectorSubcoreMesh(core_axis_name="c", subcore_axis_name="s")

    @functools.partial(
        pl.kernel,
        out_type=jax.ShapeDtypeStruct((NP, D), jnp.float32),
        mesh=mesh,
        compiler_params=pltpu.CompilerParams(use_tc_tiling_on_sc=False),
        scratch_types=[
            [pltpu.VMEM((2, _K), jnp.int32) for _ in range(_NI)],
            [pltpu.VMEM((_K, DH), jnp.float32) for _ in range(_NB)],
            pltpu.VMEM_SHARED((NP, DH), jnp.float32),
            pltpu.VMEM_SHARED((NP, DH), jnp.float32),
            [pltpu.SemaphoreType.DMA for _ in range(_NI)],
            [pltpu.SemaphoreType.DMA for _ in range(_NB)],
        ],
    )
    def agg(h_hbm, sd_hbm, out_hbm, ib, rows, hcp, acc, isems, gsems):
        c = lax.axis_index("c")
        s = lax.axis_index("s")
        base = s * rows_per_sub

        for q in range(_NI):
            pltpu.async_copy(sd_hbm.at[s, q], ib[q], isems[q])

        zero16 = jnp.zeros((16,), jnp.float32)

        def zstep(i, carry):
            rows[0][i // (DH // 16), pl.ds((i % (DH // 16)) * 16, 16)] = zero16
            return carry

        lax.fori_loop(0, _K * (DH // 16), zstep, 0)
        for t in range(n_out_chunks):
            pltpu.sync_copy(rows[0], acc.at[pl.ds(base + t * _K, _K)])
        for t in range(n_out_chunks):
            pltpu.sync_copy(h_hbm.at[pl.ds(base + t * _K, _K),
                                     pl.ds(c * DH, DH)], rows[1])
            pltpu.sync_copy(rows[1], hcp.at[pl.ds(base + t * _K, _K)])
        plsc.subcore_barrier()

        for b in range(_NB):
            pltpu.make_async_copy(sd_hbm.at[s, b], ib[b], isems[b]).wait()
            pltpu.async_copy(hcp.at[ib[b].at[0]], rows[b], gsems[b])

        def step(t, carry):
            for u in range(_NI):
                j = t * _NI + u
                rb = rows[u % _NB]
                gs = gsems[u % _NB]
                pltpu.make_async_copy(hcp.at[ib[u].at[0]], rb, gs).wait()
                pltpu.sync_copy(rb, acc.at[ib[u].at[1]], add=True)

                @pl.when(j + _NI < CH)
                def _():
                    pltpu.async_copy(sd_hbm.at[s, j + _NI], ib[u], isems[u])

                @pl.when(j + _NB < CH)
                def _():
                    q = (u + _NB) % _NI
                    pltpu.make_async_copy(sd_hbm.at[s, j + _NB], ib[q],
                                          isems[q]).wait()
                    pltpu.async_copy(hcp.at[ib[q].at[0]], rb, gs)
            return carry

        lax.fori_loop(0, CH // _NI, step, 0)
        plsc.subcore_barrier()

        for t in range(n_out_chunks):
            pltpu.sync_copy(acc.at[pl.ds(base + t * _K, _K)], rows[0])
            pltpu.sync_copy(rows[0], out_hbm.at[pl.ds(base + t * _K, _K),
                                                pl.ds(c * DH, DH)])

    return agg


def _bn_relu(z, mask, n, g, b):
    z = jnp.where(mask, z, 0.0)
    mean = jnp.sum(z, axis=0, keepdims=True) / n
    zc = jnp.where(mask, z - mean, 0.0)
    var = jnp.sum(zc * zc, axis=0, keepdims=True) / n
    zn = zc * lax.rsqrt(var + 1e-5) * g + b
    return jnp.maximum(jnp.where(mask, zn, 0.0), 0.0)


def _make_layer(NP, NR, D):
    DH = D // _NC

    def body(h_ref, agg_ref, batch_ref, w1_ref, b1_ref, g1_ref, bb1_ref,
             w2_ref, b2_ref, g2_ref, b2b_ref, out_ref, p_ref):
        mask = lax.broadcasted_iota(jnp.int32, (NP, 1), 0) < NR
        oh = (batch_ref[...] == lax.broadcasted_iota(jnp.int32, (NP, _G), 1)
              ).astype(jnp.float32)
        h = h_ref[...]
        h0 = h[:, 0:DH]
        h1 = h[:, DH:D]
        m = h + agg_ref[...]
        m0 = m[:, 0:DH]
        m1 = m[:, DH:D]
        p_ref[0] = lax.dot_general(oh, h0, (((0,), (0,)), ((), ())),
                                   preferred_element_type=jnp.float32)
        p_ref[1] = lax.dot_general(oh, h1, (((0,), (0,)), ((), ())),
                                   preferred_element_type=jnp.float32)
        z = (jnp.dot(m0, w1_ref[0:DH], preferred_element_type=jnp.float32)
             + jnp.dot(m1, w1_ref[DH:D], preferred_element_type=jnp.float32)
             + b1_ref[...])
        z = _bn_relu(z, mask, NR, g1_ref[...], bb1_ref[...])
        z = (jnp.dot(z[:, 0:DH], w2_ref[0:DH], preferred_element_type=jnp.float32)
             + jnp.dot(z[:, DH:D], w2_ref[DH:D], preferred_element_type=jnp.float32)
             + b2_ref[...])
        z = _bn_relu(z, mask, NR, g2_ref[...], b2b_ref[...])
        out_ref[...] = z

    return pl.pallas_call(
        body,
        out_shape=(jax.ShapeDtypeStruct((NP, D), jnp.float32),
                   jax.ShapeDtypeStruct((_NC, _G, DH), jnp.float32)),
    )


def _make_final(NP, D, LP, C):
    DH = D // _NC

    def body(h_ref, batch_ref, ps_ref, fcw_ref, fcb_ref, out_ref):
        oh = (batch_ref[...] == lax.broadcasted_iota(jnp.int32, (NP, _G), 1)
              ).astype(jnp.float32)
        acc = None
        for i in range(LP):
            if i < LP - 1:
                p0, p1 = ps_ref[i, 0], ps_ref[i, 1]
            else:
                p0 = lax.dot_general(oh, h_ref[:, 0:DH],
                                     (((0,), (0,)), ((), ())),
                                     preferred_element_type=jnp.float32)
                p1 = lax.dot_general(oh, h_ref[:, DH:D],
                                     (((0,), (0,)), ((), ())),
                                     preferred_element_type=jnp.float32)
            q = (jnp.dot(p0, fcw_ref[i, 0:DH], preferred_element_type=jnp.float32)
                 + jnp.dot(p1, fcw_ref[i, DH:D], preferred_element_type=jnp.float32)
                 + fcb_ref[i])
            acc = q if acc is None else acc + q
        mx = jnp.max(acc, axis=-1, keepdims=True)
        lse = jnp.log(jnp.sum(jnp.exp(acc - mx), axis=-1, keepdims=True)) + mx
        out_ref[...] = acc - lse

    return pl.pallas_call(
        body,
        out_shape=jax.ShapeDtypeStruct((_G, C), jnp.float32),
    )


def kernel(x, edge_index, batch, convW1, convb1, convg1, convbb1, convW2,
           convb2, bng, bnb, fcW, fcb):
    N, D = x.shape
    E = edge_index.shape[1]
    L = convW1.shape[0]
    C = fcW.shape[2]
    DH = D // _NC

    NP = -(-(N + 1) // (_NS * _K)) * (_NS * _K)
    CH = -(-E // (_NS * _K * _NI)) * _NI
    EP = _NS * CH * _K

    xp = jnp.concatenate([x, jnp.zeros((NP - N, D), jnp.float32)], axis=0)
    epad = jnp.full((2, EP - E), N, jnp.int32)
    ep = jnp.concatenate([edge_index, epad], axis=1)
    sd = jnp.transpose(ep.reshape(2, _NS, CH, _K), (1, 2, 0, 3))
    batch_pad = jnp.concatenate(
        [batch, jnp.full((NP - N,), _G, jnp.int32)]).reshape(NP, 1)

    agg_fn = _make_agg(NP, D, CH)
    layer_fn = _make_layer(NP, N, D)
    final_fn = _make_final(NP, D, L + 1, C)

    h = xp
    ps = []
    for i in range(L):
        ag = agg_fn(h, sd)
        h, p = layer_fn(
            h, ag, batch_pad,
            convW1[i], convb1[i].reshape(1, D), convg1[i].reshape(1, D),
            convbb1[i].reshape(1, D),
            convW2[i], convb2[i].reshape(1, D), bng[i].reshape(1, D),
            bnb[i].reshape(1, D))
        ps.append(p)
    return final_fn(h, batch_pad, jnp.stack(ps), fcW, fcb.reshape(L + 1, 1, C))


# speedup vs baseline: 1.0173x; 1.0173x over previous
import functools

import jax
import jax.numpy as jnp
from jax import lax
from jax.experimental import pallas as pl
from jax.experimental.pallas import tpu as pltpu
from jax.experimental.pallas import tpu_sc as plsc

_NC = 2
_NS = 16
_K = 128
_G = 128
_NB = 3
_NI = 6


def _make_agg(NP, D, CH):
    DH = D // _NC
    rows_per_sub = NP // _NS
    _sizes = [_K] * (rows_per_sub // _K) + (
        [rows_per_sub % _K] if rows_per_sub % _K else [])
    _offs = [i * _K for i in range(len(_sizes))]
    mesh = plsc.VectorSubcoreMesh(core_axis_name="c", subcore_axis_name="s")

    @functools.partial(
        pl.kernel,
        out_type=jax.ShapeDtypeStruct((NP, D), jnp.float32),
        mesh=mesh,
        compiler_params=pltpu.CompilerParams(use_tc_tiling_on_sc=False),
        scratch_types=[
            [pltpu.VMEM((2, _K), jnp.int32) for _ in range(_NI)],
            [pltpu.VMEM((_K, DH), jnp.float32) for _ in range(_NB)],
            pltpu.VMEM_SHARED((NP, DH), jnp.float32),
            pltpu.VMEM_SHARED((NP, DH), jnp.float32),
            [pltpu.SemaphoreType.DMA for _ in range(_NI)],
            [pltpu.SemaphoreType.DMA for _ in range(_NB)],
        ],
    )
    def agg(h_hbm, sd_hbm, out_hbm, ib, rows, hcp, acc, isems, gsems):
        c = lax.axis_index("c")
        s = lax.axis_index("s")
        base = s * rows_per_sub

        for q in range(_NI):
            pltpu.async_copy(sd_hbm.at[s, q], ib[q], isems[q])

        zero16 = jnp.zeros((16,), jnp.float32)

        def zstep(i, carry):
            rows[0][i // (DH // 16), pl.ds((i % (DH // 16)) * 16, 16)] = zero16
            return carry

        lax.fori_loop(0, _K * (DH // 16), zstep, 0)
        for off, sz in zip(_offs, _sizes):
            pltpu.sync_copy(rows[0].at[pl.ds(0, sz)],
                            acc.at[pl.ds(base + off, sz)])
        for off, sz in zip(_offs, _sizes):
            pltpu.sync_copy(h_hbm.at[pl.ds(base + off, sz),
                                     pl.ds(c * DH, DH)],
                            rows[1].at[pl.ds(0, sz)])
            pltpu.sync_copy(rows[1].at[pl.ds(0, sz)],
                            hcp.at[pl.ds(base + off, sz)])
        plsc.subcore_barrier()

        for b in range(_NB):
            pltpu.make_async_copy(sd_hbm.at[s, b], ib[b], isems[b]).wait()
            pltpu.async_copy(hcp.at[ib[b].at[0]], rows[b], gsems[b])

        def step(t, carry):
            for u in range(_NI):
                j = t * _NI + u
                rb = rows[u % _NB]
                gs = gsems[u % _NB]
                pltpu.make_async_copy(hcp.at[ib[u].at[0]], rb, gs).wait()
                pltpu.sync_copy(rb, acc.at[ib[u].at[1]], add=True)

                @pl.when(j + _NI < CH)
                def _():
                    pltpu.async_copy(sd_hbm.at[s, j + _NI], ib[u], isems[u])

                @pl.when(j + _NB < CH)
                def _():
                    q = (u + _NB) % _NI
                    pltpu.make_async_copy(sd_hbm.at[s, j + _NB], ib[q],
                                          isems[q]).wait()
                    pltpu.async_copy(hcp.at[ib[q].at[0]], rb, gs)
            return carry

        lax.fori_loop(0, CH // _NI, step, 0)
        plsc.subcore_barrier()

        for off, sz in zip(_offs, _sizes):
            pltpu.sync_copy(acc.at[pl.ds(base + off, sz)],
                            rows[0].at[pl.ds(0, sz)])
            pltpu.sync_copy(rows[0].at[pl.ds(0, sz)],
                            out_hbm.at[pl.ds(base + off, sz),
                                       pl.ds(c * DH, DH)])

    return agg


def _bn_relu(z, mask, n, g, b):
    z = jnp.where(mask, z, 0.0)
    mean = jnp.sum(z, axis=0, keepdims=True) / n
    zc = jnp.where(mask, z - mean, 0.0)
    var = jnp.sum(zc * zc, axis=0, keepdims=True) / n
    zn = zc * lax.rsqrt(var + 1e-5) * g + b
    return jnp.maximum(jnp.where(mask, zn, 0.0), 0.0)


def _make_layer(NP, NR, D):
    DH = D // _NC

    def body(h_ref, agg_ref, batch_ref, w1_ref, b1_ref, g1_ref, bb1_ref,
             w2_ref, b2_ref, g2_ref, b2b_ref, out_ref, p_ref):
        mask = lax.broadcasted_iota(jnp.int32, (NP, 1), 0) < NR
        oh = (batch_ref[...] == lax.broadcasted_iota(jnp.int32, (NP, _G), 1)
              ).astype(jnp.float32)
        h = h_ref[...]
        h0 = h[:, 0:DH]
        h1 = h[:, DH:D]
        m = h + agg_ref[...]
        m0 = m[:, 0:DH]
        m1 = m[:, DH:D]
        p_ref[0] = lax.dot_general(oh, h0, (((0,), (0,)), ((), ())),
                                   preferred_element_type=jnp.float32)
        p_ref[1] = lax.dot_general(oh, h1, (((0,), (0,)), ((), ())),
                                   preferred_element_type=jnp.float32)
        z = (jnp.dot(m0, w1_ref[0:DH], preferred_element_type=jnp.float32)
             + jnp.dot(m1, w1_ref[DH:D], preferred_element_type=jnp.float32)
             + b1_ref[...])
        z = _bn_relu(z, mask, NR, g1_ref[...], bb1_ref[...])
        z = (jnp.dot(z[:, 0:DH], w2_ref[0:DH], preferred_element_type=jnp.float32)
             + jnp.dot(z[:, DH:D], w2_ref[DH:D], preferred_element_type=jnp.float32)
             + b2_ref[...])
        z = _bn_relu(z, mask, NR, g2_ref[...], b2b_ref[...])
        out_ref[...] = z

    return pl.pallas_call(
        body,
        out_shape=(jax.ShapeDtypeStruct((NP, D), jnp.float32),
                   jax.ShapeDtypeStruct((_NC, _G, DH), jnp.float32)),
    )


def _make_final(NP, D, LP, C):
    DH = D // _NC

    def body(h_ref, batch_ref, ps_ref, fcw_ref, fcb_ref, out_ref):
        oh = (batch_ref[...] == lax.broadcasted_iota(jnp.int32, (NP, _G), 1)
              ).astype(jnp.float32)
        acc = None
        for i in range(LP):
            if i < LP - 1:
                p0, p1 = ps_ref[i, 0], ps_ref[i, 1]
            else:
                p0 = lax.dot_general(oh, h_ref[:, 0:DH],
                                     (((0,), (0,)), ((), ())),
                                     preferred_element_type=jnp.float32)
                p1 = lax.dot_general(oh, h_ref[:, DH:D],
                                     (((0,), (0,)), ((), ())),
                                     preferred_element_type=jnp.float32)
            q = (jnp.dot(p0, fcw_ref[i, 0:DH], preferred_element_type=jnp.float32)
                 + jnp.dot(p1, fcw_ref[i, DH:D], preferred_element_type=jnp.float32)
                 + fcb_ref[i])
            acc = q if acc is None else acc + q
        mx = jnp.max(acc, axis=-1, keepdims=True)
        lse = jnp.log(jnp.sum(jnp.exp(acc - mx), axis=-1, keepdims=True)) + mx
        out_ref[...] = acc - lse

    return pl.pallas_call(
        body,
        out_shape=jax.ShapeDtypeStruct((_G, C), jnp.float32),
    )


def kernel(x, edge_index, batch, convW1, convb1, convg1, convbb1, convW2,
           convb2, bng, bnb, fcW, fcb):
    N, D = x.shape
    E = edge_index.shape[1]
    L = convW1.shape[0]
    C = fcW.shape[2]
    DH = D // _NC

    NP = -(-(N + 1) // _NS) * _NS
    CH = -(-E // (_NS * _K * _NI)) * _NI
    EP = _NS * CH * _K

    xp = jnp.concatenate([x, jnp.zeros((NP - N, D), jnp.float32)], axis=0)
    epad = jnp.full((2, EP - E), N, jnp.int32)
    ep = jnp.concatenate([edge_index, epad], axis=1)
    sd = jnp.transpose(ep.reshape(2, _NS, CH, _K), (1, 2, 0, 3))
    batch_pad = jnp.concatenate(
        [batch, jnp.full((NP - N,), _G, jnp.int32)]).reshape(NP, 1)

    agg_fn = _make_agg(NP, D, CH)
    layer_fn = _make_layer(NP, N, D)
    final_fn = _make_final(NP, D, L + 1, C)

    h = xp
    ps = []
    for i in range(L):
        ag = agg_fn(h, sd)
        h, p = layer_fn(
            h, ag, batch_pad,
            convW1[i], convb1[i].reshape(1, D), convg1[i].reshape(1, D),
            convbb1[i].reshape(1, D),
            convW2[i], convb2[i].reshape(1, D), bng[i].reshape(1, D),
            bnb[i].reshape(1, D))
        ps.append(p)
    return final_fn(h, batch_pad, jnp.stack(ps), fcW, fcb.reshape(L + 1, 1, C))


# speedup vs baseline: 1.4122x; 1.3882x over previous
import functools

import jax
import jax.numpy as jnp
from jax import lax
from jax.experimental import pallas as pl
from jax.experimental.pallas import tpu as pltpu
from jax.experimental.pallas import tpu_sc as plsc

_NC = 2
_NS = 16
_K = 128
_G = 128
_NB = 3
_NI = 6


def _make_agg(NP, D, CH):
    DH = D // _NC
    rows_per_sub = NP // _NS
    _sizes = [_K] * (rows_per_sub // _K) + (
        [rows_per_sub % _K] if rows_per_sub % _K else [])
    _offs = [i * _K for i in range(len(_sizes))]
    mesh = plsc.VectorSubcoreMesh(core_axis_name="c", subcore_axis_name="s")

    @functools.partial(
        pl.kernel,
        out_type=jax.ShapeDtypeStruct((NP, D), jnp.bfloat16),
        mesh=mesh,
        compiler_params=pltpu.CompilerParams(use_tc_tiling_on_sc=False),
        scratch_types=[
            [pltpu.VMEM((2, _K), jnp.int32) for _ in range(_NI)],
            [pltpu.VMEM((_K, DH), jnp.bfloat16) for _ in range(_NB)],
            pltpu.VMEM_SHARED((NP, DH), jnp.bfloat16),
            pltpu.VMEM_SHARED((NP, DH), jnp.bfloat16),
            [pltpu.SemaphoreType.DMA for _ in range(_NI)],
            [pltpu.SemaphoreType.DMA for _ in range(_NB)],
        ],
    )
    def agg(h_hbm, sd_hbm, out_hbm, ib, rows, hcp, acc, isems, gsems):
        c = lax.axis_index("c")
        s = lax.axis_index("s")
        base = s * rows_per_sub

        for q in range(_NI):
            pltpu.async_copy(sd_hbm.at[s, q], ib[q], isems[q])

        zero32 = jnp.zeros((32,), jnp.bfloat16)

        def zstep(i, carry):
            rows[0][i // (DH // 32), pl.ds((i % (DH // 32)) * 32, 32)] = zero32
            return carry

        lax.fori_loop(0, _K * (DH // 32), zstep, 0)
        for off, sz in zip(_offs, _sizes):
            pltpu.sync_copy(rows[0].at[pl.ds(0, sz)],
                            acc.at[pl.ds(base + off, sz)])
        for off, sz in zip(_offs, _sizes):
            pltpu.sync_copy(h_hbm.at[pl.ds(base + off, sz),
                                     pl.ds(c * DH, DH)],
                            rows[1].at[pl.ds(0, sz)])
            pltpu.sync_copy(rows[1].at[pl.ds(0, sz)],
                            hcp.at[pl.ds(base + off, sz)])
        plsc.subcore_barrier()

        for b in range(_NB):
            pltpu.make_async_copy(sd_hbm.at[s, b], ib[b], isems[b]).wait()
            pltpu.async_copy(hcp.at[ib[b].at[0]], rows[b], gsems[b])

        def step(t, carry):
            for u in range(_NI):
                j = t * _NI + u
                rb = rows[u % _NB]
                gs = gsems[u % _NB]
                pltpu.make_async_copy(hcp.at[ib[u].at[0]], rb, gs).wait()
                pltpu.sync_copy(rb, acc.at[ib[u].at[1]], add=True)

                @pl.when(j + _NI < CH)
                def _():
                    pltpu.async_copy(sd_hbm.at[s, j + _NI], ib[u], isems[u])

                @pl.when(j + _NB < CH)
                def _():
                    q = (u + _NB) % _NI
                    pltpu.make_async_copy(sd_hbm.at[s, j + _NB], ib[q],
                                          isems[q]).wait()
                    pltpu.async_copy(hcp.at[ib[q].at[0]], rb, gs)
            return carry

        lax.fori_loop(0, CH // _NI, step, 0)
        plsc.subcore_barrier()

        for off, sz in zip(_offs, _sizes):
            pltpu.sync_copy(acc.at[pl.ds(base + off, sz)],
                            rows[0].at[pl.ds(0, sz)])
            pltpu.sync_copy(rows[0].at[pl.ds(0, sz)],
                            out_hbm.at[pl.ds(base + off, sz),
                                       pl.ds(c * DH, DH)])

    return agg


def _bn_relu(z, mask, n, g, b):
    z = jnp.where(mask, z, 0.0)
    mean = jnp.sum(z, axis=0, keepdims=True) / n
    zc = jnp.where(mask, z - mean, 0.0)
    var = jnp.sum(zc * zc, axis=0, keepdims=True) / n
    zn = zc * lax.rsqrt(var + 1e-5) * g + b
    return jnp.maximum(jnp.where(mask, zn, 0.0), 0.0)


def _make_layer(NP, NR, D):
    DH = D // _NC

    def body(h_ref, agg_ref, batch_ref, w1_ref, b1_ref, g1_ref, bb1_ref,
             w2_ref, b2_ref, g2_ref, b2b_ref, out_ref, hb_ref, p_ref):
        mask = lax.broadcasted_iota(jnp.int32, (NP, 1), 0) < NR
        oh = (batch_ref[...] == lax.broadcasted_iota(jnp.int32, (NP, _G), 1)
              ).astype(jnp.float32)
        h = h_ref[...]
        h0 = h[:, 0:DH]
        h1 = h[:, DH:D]
        m = h + agg_ref[...].astype(jnp.float32)
        m0 = m[:, 0:DH]
        m1 = m[:, DH:D]
        p_ref[0] = lax.dot_general(oh, h0, (((0,), (0,)), ((), ())),
                                   preferred_element_type=jnp.float32)
        p_ref[1] = lax.dot_general(oh, h1, (((0,), (0,)), ((), ())),
                                   preferred_element_type=jnp.float32)
        z = (jnp.dot(m0, w1_ref[0:DH], preferred_element_type=jnp.float32)
             + jnp.dot(m1, w1_ref[DH:D], preferred_element_type=jnp.float32)
             + b1_ref[...])
        z = _bn_relu(z, mask, NR, g1_ref[...], bb1_ref[...])
        z = (jnp.dot(z[:, 0:DH], w2_ref[0:DH], preferred_element_type=jnp.float32)
             + jnp.dot(z[:, DH:D], w2_ref[DH:D], preferred_element_type=jnp.float32)
             + b2_ref[...])
        z = _bn_relu(z, mask, NR, g2_ref[...], b2b_ref[...])
        out_ref[...] = z
        hb_ref[...] = z.astype(jnp.bfloat16)

    return pl.pallas_call(
        body,
        out_shape=(jax.ShapeDtypeStruct((NP, D), jnp.float32),
                   jax.ShapeDtypeStruct((NP, D), jnp.bfloat16),
                   jax.ShapeDtypeStruct((_NC, _G, DH), jnp.float32)),
    )


def _make_final(NP, D, LP, C):
    DH = D // _NC

    def body(h_ref, batch_ref, ps_ref, fcw_ref, fcb_ref, out_ref):
        oh = (batch_ref[...] == lax.broadcasted_iota(jnp.int32, (NP, _G), 1)
              ).astype(jnp.float32)
        acc = None
        for i in range(LP):
            if i < LP - 1:
                p0, p1 = ps_ref[i, 0], ps_ref[i, 1]
            else:
                p0 = lax.dot_general(oh, h_ref[:, 0:DH],
                                     (((0,), (0,)), ((), ())),
                                     preferred_element_type=jnp.float32)
                p1 = lax.dot_general(oh, h_ref[:, DH:D],
                                     (((0,), (0,)), ((), ())),
                                     preferred_element_type=jnp.float32)
            q = (jnp.dot(p0, fcw_ref[i, 0:DH], preferred_element_type=jnp.float32)
                 + jnp.dot(p1, fcw_ref[i, DH:D], preferred_element_type=jnp.float32)
                 + fcb_ref[i])
            acc = q if acc is None else acc + q
        mx = jnp.max(acc, axis=-1, keepdims=True)
        lse = jnp.log(jnp.sum(jnp.exp(acc - mx), axis=-1, keepdims=True)) + mx
        out_ref[...] = acc - lse

    return pl.pallas_call(
        body,
        out_shape=jax.ShapeDtypeStruct((_G, C), jnp.float32),
    )


def kernel(x, edge_index, batch, convW1, convb1, convg1, convbb1, convW2,
           convb2, bng, bnb, fcW, fcb):
    N, D = x.shape
    E = edge_index.shape[1]
    L = convW1.shape[0]
    C = fcW.shape[2]
    DH = D // _NC

    NP = -(-(N + 1) // _NS) * _NS
    CH = -(-E // (_NS * _K * _NI)) * _NI
    EP = _NS * CH * _K

    xp = jnp.concatenate([x, jnp.zeros((NP - N, D), jnp.float32)], axis=0)
    epad = jnp.full((2, EP - E), N, jnp.int32)
    ep = jnp.concatenate([edge_index, epad], axis=1)
    sd = jnp.transpose(ep.reshape(2, _NS, CH, _K), (1, 2, 0, 3))
    batch_pad = jnp.concatenate(
        [batch, jnp.full((NP - N,), _G, jnp.int32)]).reshape(NP, 1)

    agg_fn = _make_agg(NP, D, CH)
    layer_fn = _make_layer(NP, N, D)
    final_fn = _make_final(NP, D, L + 1, C)

    h = xp
    hb = xp.astype(jnp.bfloat16)
    ps = []
    for i in range(L):
        ag = agg_fn(hb, sd)
        h, hb, p = layer_fn(
            h, ag, batch_pad,
            convW1[i], convb1[i].reshape(1, D), convg1[i].reshape(1, D),
            convbb1[i].reshape(1, D),
            convW2[i], convb2[i].reshape(1, D), bng[i].reshape(1, D),
            bnb[i].reshape(1, D))
        ps.append(p)
    return final_fn(h, batch_pad, jnp.stack(ps), fcW, fcb.reshape(L + 1, 1, C))
